# Initial kernel scaffold; baseline (speedup 1.0000x reference)
import jax
import jax.numpy as jnp
from jax.experimental import pallas as pl


def kernel(length_q, length_k, embeddings_table):
    raise NotImplementedError("write your pallas kernel here")


# speedup vs baseline: 6.4262x; 6.4262x over previous
import jax
import jax.numpy as jnp
from jax import lax
from jax.experimental import pallas as pl
from jax.experimental.pallas import tpu as pltpu
from jax.experimental.pallas import tpu_sc as plsc

_NUM_UNITS = 64
_MAX_REL = 128
_LQ = 2048
_LK = 2048

_NC = 2
_NS = 16
_NW = _NC * _NS

_R = _LQ // _NW
_CK = 1024
_SROWS = _CK + _R - 1


def _band_expand(v_hbm, out_hbm, stage):
  wid = lax.axis_index("s") * _NC + lax.axis_index("c")
  q0 = wid * _R
  for kb in range(_LK // _CK):
    k0 = kb * _CK
    start = (_LK - 1) - (q0 + _R - 1) + k0
    pltpu.sync_copy(v_hbm.at[pl.ds(start, _SROWS)], stage)

    def body(r, carry):
      src = stage.at[pl.ds(_R - 1 - r, _CK)]
      dst = out_hbm.at[pl.ds((q0 + r) * _LK + k0, _CK)]
      pltpu.sync_copy(src, dst)
      return carry

    lax.fori_loop(0, _R, body, 0)


def kernel(length_q, length_k, embeddings_table):
  t = embeddings_table.astype(jnp.float32)
  v = jnp.concatenate(
      [
          jnp.broadcast_to(t[0:1], (_LK - 1 - _MAX_REL, _NUM_UNITS)),
          t,
          jnp.broadcast_to(t[2 * _MAX_REL:, :], (_LQ - 1 - _MAX_REL, _NUM_UNITS)),
      ],
      axis=0,
  )

  mesh = plsc.VectorSubcoreMesh(
      core_axis_name="c", subcore_axis_name="s", num_cores=_NC, num_subcores=_NS
  )
  out = pl.kernel(
      _band_expand,
      out_type=jax.ShapeDtypeStruct((_LQ * _LK, _NUM_UNITS), jnp.float32),
      mesh=mesh,
      scratch_types=[pltpu.VMEM((_SROWS, _NUM_UNITS), jnp.float32)],
      compiler_params=pltpu.CompilerParams(use_tc_tiling_on_sc=False),
  )(v)
  return out.reshape(_LQ, _LK, _NUM_UNITS)
